# baseline (speedup 1.0000x reference)
import functools

import jax
import jax.numpy as jnp
from jax import lax
from jax.experimental import pallas as pl
from jax.experimental.pallas import tpu as pltpu
from jax.experimental.pallas import tpu_sc as plsc

N_ATOMS_K = 262144
N_SPECIES_K = 118
DIM_K = 256
KEY_PAD = 128
LANES = 16
NUM_WORKERS = 32
B_PER_W = N_ATOMS_K // NUM_WORKERS
CHUNK = 128
N_CHUNKS = B_PER_W // CHUNK
UNROLL = 4


def _sc_lookup_kernel(zs_hbm, zk_hbm, values_hbm, out_hbm,
                      zs_v, zk_v, inv_v, idx_v, table_v,
                      rows0_v, rows1_v, wsem0, wsem1):
    wid = lax.axis_index("s") * 2 + lax.axis_index("c")
    base = wid * B_PER_W

    pltpu.sync_copy(zs_hbm.at[pl.ds(base, B_PER_W)], zs_v)
    pltpu.sync_copy(zk_hbm, zk_v)
    pltpu.sync_copy(values_hbm, table_v)

    lanes = lax.iota(jnp.int32, LANES)
    for j in range(KEY_PAD // LANES):
        keys = zk_v[pl.ds(j * LANES, LANES)]
        plsc.store_scatter(inv_v, [keys], lanes + j * LANES)

    def map_body(i, carry):
        z = zs_v[pl.ds(i * LANES, LANES)]
        idx_v[pl.ds(i * LANES, LANES)] = plsc.load_gather(inv_v, [z])
        return carry

    lax.fori_loop(0, B_PER_W // LANES, map_body, 0)

    bufs = (rows0_v, rows1_v)
    wsems = (wsem0, wsem1)

    def out_at(g):
        return out_hbm.at[pl.ds(base + g * CHUNK, CHUNK)]

    def fill(g, buf):
        def fill_body(i, carry):
            zv = idx_v[pl.ds(g * CHUNK + i * LANES, LANES)]
            for u in range(LANES):
                a = i * LANES + u
                z = zv[u]
                for c in range(DIM_K // LANES):
                    buf[a, pl.ds(c * LANES, LANES)] = (
                        table_v[z, pl.ds(c * LANES, LANES)])
            return carry

        lax.fori_loop(0, CHUNK // LANES, fill_body, 0)

    fill(0, bufs[0])

    def ring_body(h, carry):
        for b in range(2):
            g = 2 * h + b
            pltpu.async_copy(bufs[b], out_at(g), wsems[b])

            @pl.when(g + 1 < N_CHUNKS)
            def _():
                @pl.when(g >= 1)
                def _():
                    pltpu.make_async_copy(bufs[1 - b], out_at(g - 1),
                                          wsems[1 - b]).wait()
                fill(g + 1, bufs[1 - b])
        return carry

    lax.fori_loop(0, N_CHUNKS // 2, ring_body, 0)

    for g in (N_CHUNKS - 2, N_CHUNKS - 1):
        pltpu.make_async_copy(bufs[g % 2], out_at(g), wsems[g % 2]).wait()


@jax.jit
def kernel(Zs, Z_keys, values):
    n_keys = Z_keys.shape[0]
    zk_pad = jnp.concatenate(
        [Z_keys.astype(jnp.int32),
         jnp.arange(n_keys, KEY_PAD, dtype=jnp.int32)])
    values_pad = jnp.pad(values, ((0, KEY_PAD - n_keys), (0, 0)))
    mesh = plsc.VectorSubcoreMesh(core_axis_name="c", subcore_axis_name="s")
    run = pl.kernel(
        _sc_lookup_kernel,
        mesh=mesh,
        compiler_params=pltpu.CompilerParams(needs_layout_passes=False),
        out_type=jax.ShapeDtypeStruct((N_ATOMS_K, DIM_K), jnp.float32),
        scratch_types=[
            pltpu.VMEM((B_PER_W,), jnp.int32),
            pltpu.VMEM((KEY_PAD,), jnp.int32),
            pltpu.VMEM((KEY_PAD,), jnp.int32),
            pltpu.VMEM((B_PER_W,), jnp.int32),
            pltpu.VMEM((KEY_PAD, DIM_K), jnp.float32),
            pltpu.VMEM((CHUNK, DIM_K), jnp.float32),
            pltpu.VMEM((CHUNK, DIM_K), jnp.float32),
            pltpu.SemaphoreType.DMA,
            pltpu.SemaphoreType.DMA,
        ],
    )
    return run(Zs, zk_pad, values_pad)


# speedup vs baseline: 2.6035x; 2.6035x over previous
import functools

import jax
import jax.numpy as jnp
from jax import lax
from jax.experimental import pallas as pl
from jax.experimental.pallas import tpu as pltpu
from jax.experimental.pallas import tpu_sc as plsc

N_ATOMS_K = 262144
N_SPECIES_K = 118
DIM_K = 256
KEY_PAD = 128
LANES = 16
NUM_WORKERS = 32
B_PER_W = N_ATOMS_K // NUM_WORKERS
CHUNK = 128
N_CHUNKS = B_PER_W // CHUNK
UNROLL = 4


def _sc_lookup_kernel(zs_hbm, zk_hbm, values_hbm, out_hbm,
                      zs_v, zk_v, inv_v, idx_v, table_v,
                      rows0_v, rows1_v, wsem0, wsem1):
    wid = lax.axis_index("s") * 2 + lax.axis_index("c")
    base = wid * B_PER_W

    pltpu.sync_copy(zs_hbm.at[pl.ds(base, B_PER_W)], zs_v)
    pltpu.sync_copy(zk_hbm, zk_v)
    pltpu.sync_copy(values_hbm, table_v)

    lanes = lax.iota(jnp.int32, LANES)
    for j in range(KEY_PAD // LANES):
        keys = zk_v[pl.ds(j * LANES, LANES)]
        plsc.store_scatter(inv_v, [keys], lanes + j * LANES)

    def map_body(i, carry):
        z = zs_v[pl.ds(i * LANES, LANES)]
        idx_v[pl.ds(i * LANES, LANES)] = plsc.load_gather(inv_v, [z])
        return carry

    lax.fori_loop(0, B_PER_W // LANES, map_body, 0)

    bufs = (rows0_v, rows1_v)
    wsems = (wsem0, wsem1)

    def out_at(g):
        return out_hbm.at[pl.ds(base + g * CHUNK, CHUNK)]

    def fill(g, buf):
        def fill_body(i, carry):
            zv = idx_v[pl.ds(g * CHUNK + i * LANES, LANES)]
            for u in range(0, LANES, 2):
                a = i * LANES + u
                z0 = zv[u]
                z1 = zv[u + 1]
                vals0 = [table_v[z0, pl.ds(c * LANES, LANES)]
                         for c in range(DIM_K // LANES)]
                vals1 = [table_v[z1, pl.ds(c * LANES, LANES)]
                         for c in range(DIM_K // LANES)]
                for c in range(DIM_K // LANES):
                    buf[a, pl.ds(c * LANES, LANES)] = vals0[c]
                for c in range(DIM_K // LANES):
                    buf[a + 1, pl.ds(c * LANES, LANES)] = vals1[c]
            return carry

        lax.fori_loop(0, CHUNK // LANES, fill_body, 0)

    fill(0, bufs[0])

    def ring_body(h, carry):
        for b in range(2):
            g = 2 * h + b
            pltpu.async_copy(bufs[b], out_at(g), wsems[b])

            @pl.when(g + 1 < N_CHUNKS)
            def _():
                @pl.when(g >= 1)
                def _():
                    pltpu.make_async_copy(bufs[1 - b], out_at(g - 1),
                                          wsems[1 - b]).wait()
                fill(g + 1, bufs[1 - b])
        return carry

    lax.fori_loop(0, N_CHUNKS // 2, ring_body, 0)

    for g in (N_CHUNKS - 2, N_CHUNKS - 1):
        pltpu.make_async_copy(bufs[g % 2], out_at(g), wsems[g % 2]).wait()


@jax.jit
def kernel(Zs, Z_keys, values):
    n_keys = Z_keys.shape[0]
    zk_pad = jnp.concatenate(
        [Z_keys.astype(jnp.int32),
         jnp.arange(n_keys, KEY_PAD, dtype=jnp.int32)])
    values_pad = jnp.pad(values, ((0, KEY_PAD - n_keys), (0, 0)))
    mesh = plsc.VectorSubcoreMesh(core_axis_name="c", subcore_axis_name="s")
    run = pl.kernel(
        _sc_lookup_kernel,
        mesh=mesh,
        compiler_params=pltpu.CompilerParams(needs_layout_passes=False),
        out_type=jax.ShapeDtypeStruct((N_ATOMS_K, DIM_K), jnp.float32),
        scratch_types=[
            pltpu.VMEM((B_PER_W,), jnp.int32),
            pltpu.VMEM((KEY_PAD,), jnp.int32),
            pltpu.VMEM((KEY_PAD,), jnp.int32),
            pltpu.VMEM((B_PER_W,), jnp.int32),
            pltpu.VMEM((KEY_PAD, DIM_K), jnp.float32),
            pltpu.VMEM((CHUNK, DIM_K), jnp.float32),
            pltpu.VMEM((CHUNK, DIM_K), jnp.float32),
            pltpu.SemaphoreType.DMA,
            pltpu.SemaphoreType.DMA,
        ],
    )
    return run(Zs, zk_pad, values_pad)


# speedup vs baseline: 3.0902x; 1.1870x over previous
import functools

import jax
import jax.numpy as jnp
from jax import lax
from jax.experimental import pallas as pl
from jax.experimental.pallas import tpu as pltpu
from jax.experimental.pallas import tpu_sc as plsc

N_ATOMS_K = 262144
N_SPECIES_K = 118
DIM_K = 256
KEY_PAD = 128
LANES = 16
NUM_WORKERS = 32
B_PER_W = N_ATOMS_K // NUM_WORKERS
CHUNK = 128
N_CHUNKS = B_PER_W // CHUNK
GATH = 32


def _sc_lookup_kernel(zs_hbm, zk_hbm, rep_hbm, out_hbm,
                      zs_v, zk_v, inv_v, idx_v, idxg_v, table_v,
                      rows0_v, rows1_v, wsem0, wsem1, gsem0, gsem1):
    wid = lax.axis_index("s") * 2 + lax.axis_index("c")
    base = wid * B_PER_W

    pltpu.sync_copy(zs_hbm.at[pl.ds(base, B_PER_W)], zs_v)
    pltpu.sync_copy(zk_hbm, zk_v)
    pltpu.sync_copy(rep_hbm.at[pl.ds(wid * KEY_PAD, KEY_PAD)], table_v)

    lanes = lax.iota(jnp.int32, LANES)
    for j in range(KEY_PAD // LANES):
        keys = zk_v[pl.ds(j * LANES, LANES)]
        plsc.store_scatter(inv_v, [keys], lanes + j * LANES)

    def map_body(i, carry):
        z = zs_v[pl.ds(i * LANES, LANES)]
        idx = plsc.load_gather(inv_v, [z])
        idx_v[pl.ds(i * LANES, LANES)] = idx
        idxg_v[pl.ds(i * LANES, LANES)] = idx + wid * KEY_PAD
        return carry

    lax.fori_loop(0, B_PER_W // LANES, map_body, 0)

    bufs = (rows0_v, rows1_v)
    wsems = (wsem0, wsem1)
    gsems = (gsem0, gsem1)

    def out_at(g):
        return out_hbm.at[pl.ds(base + g * CHUNK, CHUNK)]

    def _gather_copy(g, b):
        return pltpu.make_async_copy(
            rep_hbm.at[idxg_v.at[pl.ds(g * CHUNK, GATH)]],
            bufs[b].at[pl.ds(0, GATH)], gsems[b])

    def gather(g, b):
        _gather_copy(g, b).start()

    def gather_wait(g, b):
        _gather_copy(g, b).wait()

    def fill(g, buf):
        def fill_body(i, carry):
            zv = idx_v[pl.ds(g * CHUNK + GATH + i * LANES, LANES)]
            for u in range(0, LANES, 2):
                a = GATH + i * LANES + u
                z0 = zv[u]
                z1 = zv[u + 1]
                vals0 = [table_v[z0, pl.ds(c * LANES, LANES)]
                         for c in range(DIM_K // LANES)]
                vals1 = [table_v[z1, pl.ds(c * LANES, LANES)]
                         for c in range(DIM_K // LANES)]
                for c in range(DIM_K // LANES):
                    buf[a, pl.ds(c * LANES, LANES)] = vals0[c]
                for c in range(DIM_K // LANES):
                    buf[a + 1, pl.ds(c * LANES, LANES)] = vals1[c]
            return carry

        lax.fori_loop(0, (CHUNK - GATH) // LANES, fill_body, 0)

    gather(0, 0)
    fill(0, bufs[0])

    def ring_body(h, carry):
        for b in range(2):
            g = 2 * h + b
            gather_wait(g, b)
            pltpu.async_copy(bufs[b], out_at(g), wsems[b])

            @pl.when(g + 1 < N_CHUNKS)
            def _():
                @pl.when(g >= 1)
                def _():
                    pltpu.make_async_copy(bufs[1 - b], out_at(g - 1),
                                          wsems[1 - b]).wait()
                gather(g + 1, 1 - b)
                fill(g + 1, bufs[1 - b])
        return carry

    lax.fori_loop(0, N_CHUNKS // 2, ring_body, 0)

    for g in (N_CHUNKS - 2, N_CHUNKS - 1):
        pltpu.make_async_copy(bufs[g % 2], out_at(g), wsems[g % 2]).wait()


@jax.jit
def kernel(Zs, Z_keys, values):
    n_keys = Z_keys.shape[0]
    zk_pad = jnp.concatenate(
        [Z_keys.astype(jnp.int32),
         jnp.arange(n_keys, KEY_PAD, dtype=jnp.int32)])
    values_pad = jnp.pad(values, ((0, KEY_PAD - n_keys), (0, 0)))
    mesh = plsc.VectorSubcoreMesh(core_axis_name="c", subcore_axis_name="s")
    run = pl.kernel(
        _sc_lookup_kernel,
        mesh=mesh,
        compiler_params=pltpu.CompilerParams(needs_layout_passes=False),
        out_type=jax.ShapeDtypeStruct((N_ATOMS_K, DIM_K), jnp.float32),
        scratch_types=[
            pltpu.VMEM((B_PER_W,), jnp.int32),
            pltpu.VMEM((KEY_PAD,), jnp.int32),
            pltpu.VMEM((KEY_PAD,), jnp.int32),
            pltpu.VMEM((B_PER_W,), jnp.int32),
            pltpu.VMEM((B_PER_W,), jnp.int32),
            pltpu.VMEM((KEY_PAD, DIM_K), jnp.float32),
            pltpu.VMEM((CHUNK, DIM_K), jnp.float32),
            pltpu.VMEM((CHUNK, DIM_K), jnp.float32),
            pltpu.SemaphoreType.DMA,
            pltpu.SemaphoreType.DMA,
            pltpu.SemaphoreType.DMA,
            pltpu.SemaphoreType.DMA,
        ],
    )
    values_rep = jnp.tile(values_pad, (NUM_WORKERS, 1))
    return run(Zs, zk_pad, values_rep)
